# baseline (speedup 1.0000x reference)
import functools

import jax
from jax import lax
import jax.numpy as jnp
from jax.experimental import pallas as pl
from jax.experimental.pallas import tpu as pltpu
from jax.experimental.pallas import tpu_sc as plsc

VOCAB = 100000
D = 32
B = 1024
CTX = 20
NC, NS = 2, 16
NW = NC * NS
TV = 2048


def _sc_gather(emb, idx):
    n = idx.shape[0]
    per_w = n // NW
    mesh = plsc.VectorSubcoreMesh(core_axis_name="c", subcore_axis_name="s")

    @functools.partial(
        pl.kernel, mesh=mesh,
        compiler_params=pltpu.CompilerParams(use_tc_tiling_on_sc=False),
        out_type=jax.ShapeDtypeStruct((n, D), emb.dtype),
        scratch_types=[
            pltpu.VMEM((per_w,), jnp.int32),
            pltpu.VMEM((per_w, D), jnp.float32),
            pltpu.SemaphoreType.DMA,
        ],
    )
    def gather_kernel(emb_hbm, idx_hbm, out_hbm, idx_v, rows_v, sem):
        wid = lax.axis_index("s") * NC + lax.axis_index("c")
        base = wid * per_w
        pltpu.sync_copy(idx_hbm.at[pl.ds(base, per_w)], idx_v)
        pltpu.async_copy(emb_hbm.at[idx_v], rows_v, sem).wait()
        pltpu.sync_copy(rows_v, out_hbm.at[pl.ds(base, per_w)])

    return gather_kernel(emb, idx)


def _proj_kernel(g_ref, p_ref, w_ref, b_ref, out_ref, pooled_ref):
    @pl.when(pl.program_id(0) == 0)
    def _():
        pooled_ref[...] = jax.lax.dot_general(
            g_ref[...], p_ref[...], (((1,), (0,)), ((), ())),
            preferred_element_type=jnp.float32,
            precision=jax.lax.Precision.HIGHEST)
    out_ref[...] = jax.lax.dot_general(
        pooled_ref[...], w_ref[...], (((1,), (1,)), ((), ())),
        preferred_element_type=jnp.float32,
        precision=jax.lax.Precision.HIGHEST) + b_ref[...]


def kernel(inputs, emb, W, b):
    idx = inputs.reshape(B * CTX).astype(jnp.int32)
    gathered = _sc_gather(emb, idx)
    g2d = gathered.reshape(B, CTX * D)
    pool = jnp.tile(jnp.eye(D, dtype=jnp.float32), (CTX, 1)) / CTX
    b2d = b.reshape(1, VOCAB)
    nv = pl.cdiv(VOCAB, TV)
    logits = pl.pallas_call(
        _proj_kernel,
        grid=(nv,),
        in_specs=[
            pl.BlockSpec((B, CTX * D), lambda i: (0, 0)),
            pl.BlockSpec((CTX * D, D), lambda i: (0, 0)),
            pl.BlockSpec((TV, D), lambda i: (i, 0)),
            pl.BlockSpec((1, TV), lambda i: (0, i)),
        ],
        out_specs=pl.BlockSpec((B, TV), lambda i: (0, i)),
        out_shape=jax.ShapeDtypeStruct((B, VOCAB), jnp.float32),
        scratch_shapes=[pltpu.VMEM((B, D), jnp.float32)],
    )(g2d, pool, W, b2d)
    return logits


# speedup vs baseline: 1.3163x; 1.3163x over previous
import functools

import jax
from jax import lax
import jax.numpy as jnp
from jax.experimental import pallas as pl
from jax.experimental.pallas import tpu as pltpu
from jax.experimental.pallas import tpu_sc as plsc

VOCAB = 100000
D = 32
B = 1024
CTX = 20
NC, NS = 2, 16
NW = NC * NS
TV = 2048


def _sc_gather(emb, idx):
    n = idx.shape[0]
    per_w = n // NW
    mesh = plsc.VectorSubcoreMesh(core_axis_name="c", subcore_axis_name="s")

    @functools.partial(
        pl.kernel, mesh=mesh,
        compiler_params=pltpu.CompilerParams(use_tc_tiling_on_sc=False),
        out_type=jax.ShapeDtypeStruct((n, D), emb.dtype),
        scratch_types=[
            pltpu.VMEM((per_w,), jnp.int32),
            pltpu.VMEM((per_w, D), jnp.float32),
            pltpu.SemaphoreType.DMA,
        ],
    )
    def gather_kernel(emb_hbm, idx_hbm, out_hbm, idx_v, rows_v, sem):
        wid = lax.axis_index("s") * NC + lax.axis_index("c")
        base = wid * per_w
        pltpu.sync_copy(idx_hbm.at[pl.ds(base, per_w)], idx_v)
        pltpu.async_copy(emb_hbm.at[idx_v], rows_v, sem).wait()
        pltpu.sync_copy(rows_v, out_hbm.at[pl.ds(base, per_w)])

    return gather_kernel(emb, idx)


def _proj_kernel(g_ref, p_ref, w_ref, b_ref, out_ref, pooled_ref):
    @pl.when(pl.program_id(0) == 0)
    def _():
        pooled_ref[...] = jax.lax.dot_general(
            g_ref[...], p_ref[...], (((1,), (0,)), ((), ())),
            preferred_element_type=jnp.float32,
            precision=jax.lax.Precision.HIGHEST)
    out_ref[...] = jax.lax.dot_general(
        pooled_ref[...], w_ref[...], (((1,), (1,)), ((), ())),
        preferred_element_type=jnp.float32,
        precision=jax.lax.Precision.DEFAULT) + b_ref[...]


def kernel(inputs, emb, W, b):
    idx = inputs.reshape(B * CTX).astype(jnp.int32)
    gathered = _sc_gather(emb, idx)
    g2d = gathered.reshape(B, CTX * D)
    pool = jnp.tile(jnp.eye(D, dtype=jnp.float32), (CTX, 1)) / CTX
    b2d = b.reshape(1, VOCAB)
    nv = pl.cdiv(VOCAB, TV)
    logits = pl.pallas_call(
        _proj_kernel,
        grid=(nv,),
        in_specs=[
            pl.BlockSpec((B, CTX * D), lambda i: (0, 0)),
            pl.BlockSpec((CTX * D, D), lambda i: (0, 0)),
            pl.BlockSpec((TV, D), lambda i: (i, 0)),
            pl.BlockSpec((1, TV), lambda i: (0, i)),
        ],
        out_specs=pl.BlockSpec((B, TV), lambda i: (0, i)),
        out_shape=jax.ShapeDtypeStruct((B, VOCAB), jnp.float32),
        scratch_shapes=[pltpu.VMEM((B, D), jnp.float32)],
    )(g2d, pool, W, b2d)
    return logits


# speedup vs baseline: 1.3351x; 1.0143x over previous
import functools

import jax
from jax import lax
import jax.numpy as jnp
from jax.experimental import pallas as pl
from jax.experimental.pallas import tpu as pltpu
from jax.experimental.pallas import tpu_sc as plsc

VOCAB = 100000
D = 32
B = 1024
CTX = 20
NC, NS = 2, 16
NW = NC * NS
TV = 2048


def _sc_gather(emb, idx):
    n = idx.shape[0]
    per_w = n // NW
    mesh = plsc.VectorSubcoreMesh(core_axis_name="c", subcore_axis_name="s")

    @functools.partial(
        pl.kernel, mesh=mesh,
        compiler_params=pltpu.CompilerParams(use_tc_tiling_on_sc=False),
        out_type=jax.ShapeDtypeStruct((n, D), emb.dtype),
        scratch_types=[
            pltpu.VMEM((per_w,), jnp.int32),
            pltpu.VMEM((per_w, D), jnp.float32),
            pltpu.SemaphoreType.DMA,
        ],
    )
    def gather_kernel(emb_hbm, idx_hbm, out_hbm, idx_v, rows_v, sem):
        wid = lax.axis_index("s") * NC + lax.axis_index("c")
        base = wid * per_w
        pltpu.sync_copy(idx_hbm.at[pl.ds(base, per_w)], idx_v)
        pltpu.async_copy(emb_hbm.at[idx_v], rows_v, sem).wait()
        pltpu.sync_copy(rows_v, out_hbm.at[pl.ds(base, per_w)])

    return gather_kernel(emb, idx)


def _proj_kernel(g_ref, p_ref, w_ref, b_ref, out_ref, pooled_ref):
    @pl.when(pl.program_id(0) == 0)
    def _():
        pooled_ref[...] = jax.lax.dot_general(
            g_ref[...], p_ref[...], (((1,), (0,)), ((), ())),
            preferred_element_type=jnp.float32,
            precision=jax.lax.Precision.HIGHEST).astype(jnp.bfloat16)
    out_ref[...] = jax.lax.dot_general(
        pooled_ref[...], w_ref[...], (((1,), (1,)), ((), ())),
        preferred_element_type=jnp.float32) + b_ref[...]


def kernel(inputs, emb, W, b):
    idx = inputs.reshape(B * CTX).astype(jnp.int32)
    gathered = _sc_gather(emb, idx)
    g2d = gathered.reshape(B, CTX * D)
    pool = jnp.tile(jnp.eye(D, dtype=jnp.float32), (CTX, 1)) / CTX
    Wh = W.astype(jnp.bfloat16)
    b2d = b.reshape(1, VOCAB)
    nv = pl.cdiv(VOCAB, TV)
    logits = pl.pallas_call(
        _proj_kernel,
        grid=(nv,),
        in_specs=[
            pl.BlockSpec((B, CTX * D), lambda i: (0, 0)),
            pl.BlockSpec((CTX * D, D), lambda i: (0, 0)),
            pl.BlockSpec((TV, D), lambda i: (i, 0)),
            pl.BlockSpec((1, TV), lambda i: (0, i)),
        ],
        out_specs=pl.BlockSpec((B, TV), lambda i: (0, i)),
        out_shape=jax.ShapeDtypeStruct((B, VOCAB), jnp.float32),
        scratch_shapes=[pltpu.VMEM((B, D), jnp.bfloat16)],
    )(g2d, pool, Wh, b2d)
    return logits


# speedup vs baseline: 3.6399x; 2.7264x over previous
import functools

import jax
from jax import lax
import jax.numpy as jnp
import numpy as np
from jax.experimental import pallas as pl
from jax.experimental.pallas import tpu as pltpu
from jax.experimental.pallas import tpu_sc as plsc

VOCAB = 100000
D = 32
B = 1024
CTX = 20
NC, NS = 2, 16
NW = NC * NS
TV = 2048

_POOL = np.tile(np.eye(D, dtype=np.float32), (CTX, 1)) / CTX


def _sc_gather(emb, idx):
    n = idx.shape[0]
    per_w = n // NW
    mesh = plsc.VectorSubcoreMesh(core_axis_name="c", subcore_axis_name="s")

    @functools.partial(
        pl.kernel, mesh=mesh,
        compiler_params=pltpu.CompilerParams(use_tc_tiling_on_sc=False),
        out_type=jax.ShapeDtypeStruct((n, D), emb.dtype),
        scratch_types=[
            pltpu.VMEM((per_w,), jnp.int32),
            pltpu.VMEM((per_w, D), jnp.float32),
            pltpu.SemaphoreType.DMA,
        ],
    )
    def gather_kernel(emb_hbm, idx_hbm, out_hbm, idx_v, rows_v, sem):
        wid = lax.axis_index("s") * NC + lax.axis_index("c")
        base = wid * per_w
        pltpu.sync_copy(idx_hbm.at[pl.ds(base, per_w)], idx_v)
        pltpu.async_copy(emb_hbm.at[idx_v], rows_v, sem).wait()
        pltpu.sync_copy(rows_v, out_hbm.at[pl.ds(base, per_w)])

    return gather_kernel(emb, idx)


def _proj_kernel(g_ref, p_ref, w_ref, out_ref, pooled_ref):
    @pl.when(pl.program_id(0) == 0)
    def _():
        pooled_ref[:, :D] = jax.lax.dot_general(
            g_ref[...], p_ref[...], (((1,), (0,)), ((), ())),
            preferred_element_type=jnp.float32,
            precision=jax.lax.Precision.HIGHEST).astype(jnp.bfloat16)
        pooled_ref[:, D:] = jnp.ones((B, 1), jnp.bfloat16)
    out_ref[...] = jax.lax.dot_general(
        w_ref[...], pooled_ref[...], (((0,), (1,)), ((), ())),
        preferred_element_type=jnp.float32)


def kernel(inputs, emb, W, b):
    idx = inputs.reshape(B * CTX).astype(jnp.int32)
    gathered = _sc_gather(emb, idx)
    g2d = gathered.reshape(B, CTX * D)
    pool = jnp.asarray(_POOL)
    w_aug = jnp.concatenate([W.T, b[None, :]], axis=0).astype(jnp.bfloat16)
    nv = pl.cdiv(VOCAB, TV)
    out_t = pl.pallas_call(
        _proj_kernel,
        grid=(nv,),
        in_specs=[
            pl.BlockSpec((B, CTX * D), lambda i: (0, 0)),
            pl.BlockSpec((CTX * D, D), lambda i: (0, 0)),
            pl.BlockSpec((D + 1, TV), lambda i: (0, i)),
        ],
        out_specs=pl.BlockSpec((TV, B), lambda i: (i, 0)),
        out_shape=jax.ShapeDtypeStruct((VOCAB, B), jnp.float32),
        scratch_shapes=[pltpu.VMEM((B, D + 1), jnp.bfloat16)],
    )(g2d, pool, w_aug)
    return out_t.T


# speedup vs baseline: 3.6719x; 1.0088x over previous
import functools

import jax
from jax import lax
import jax.numpy as jnp
import numpy as np
from jax.experimental import pallas as pl
from jax.experimental.pallas import tpu as pltpu
from jax.experimental.pallas import tpu_sc as plsc

VOCAB = 100000
D = 32
DP = 128
B = 1024
CTX = 20
NC, NS = 2, 16
NW = NC * NS
TV = 2048

_PSEL = np.zeros((DP, D), dtype=np.float32)
_PSEL[:D, :D] = np.eye(D, dtype=np.float32)


def _sc_gather(emb128, idx):
    n = idx.shape[0]
    per_w = n // NW
    mesh = plsc.VectorSubcoreMesh(core_axis_name="c", subcore_axis_name="s")

    @functools.partial(
        pl.kernel, mesh=mesh,
        out_type=jax.ShapeDtypeStruct((n, DP), emb128.dtype),
        scratch_types=[
            pltpu.VMEM((per_w,), jnp.int32),
            pltpu.VMEM((per_w, DP), jnp.float32),
            pltpu.SemaphoreType.DMA,
        ],
    )
    def gather_kernel(emb_hbm, idx_hbm, out_hbm, idx_v, rows_v, sem):
        wid = lax.axis_index("s") * NC + lax.axis_index("c")
        base = wid * per_w
        pltpu.sync_copy(idx_hbm.at[pl.ds(base, per_w)], idx_v)
        pltpu.async_copy(emb_hbm.at[idx_v], rows_v, sem).wait()
        pltpu.sync_copy(rows_v, out_hbm.at[pl.ds(base, per_w)])

    return gather_kernel(emb128, idx)


def _proj_kernel(g_ref, p_ref, w_ref, out_ref, pooled_ref):
    @pl.when(pl.program_id(0) == 0)
    def _():
        acc = g_ref[0]
        for c in range(1, CTX):
            acc = acc + g_ref[c]
        pooled_ref[:, :D] = (jax.lax.dot_general(
            acc, p_ref[...], (((1,), (0,)), ((), ())),
            preferred_element_type=jnp.float32,
            precision=jax.lax.Precision.HIGHEST) / CTX).astype(jnp.bfloat16)
        pooled_ref[:, D:] = jnp.ones((B, 1), jnp.bfloat16)
    out_ref[...] = jax.lax.dot_general(
        w_ref[...], pooled_ref[...], (((0,), (1,)), ((), ())),
        preferred_element_type=jnp.float32)


def kernel(inputs, emb, W, b):
    idx = inputs.T.reshape(B * CTX).astype(jnp.int32)
    emb128 = jnp.pad(emb, ((0, 0), (0, DP - D)))
    gathered = _sc_gather(emb128, idx)
    g3 = gathered.reshape(CTX, B, DP)
    psel = jnp.asarray(_PSEL)
    w_aug = jnp.concatenate([W.T, b[None, :]], axis=0).astype(jnp.bfloat16)
    nv = pl.cdiv(VOCAB, TV)
    out_t = pl.pallas_call(
        _proj_kernel,
        grid=(nv,),
        in_specs=[
            pl.BlockSpec((CTX, B, DP), lambda i: (0, 0, 0)),
            pl.BlockSpec((DP, D), lambda i: (0, 0)),
            pl.BlockSpec((D + 1, TV), lambda i: (0, i)),
        ],
        out_specs=pl.BlockSpec((TV, B), lambda i: (i, 0)),
        out_shape=jax.ShapeDtypeStruct((VOCAB, B), jnp.float32),
        scratch_shapes=[pltpu.VMEM((B, D + 1), jnp.bfloat16)],
    )(g3, psel, w_aug)
    return out_t.T


# speedup vs baseline: 3.8524x; 1.0492x over previous
import functools

import jax
from jax import lax
import jax.numpy as jnp
import numpy as np
from jax.experimental import pallas as pl
from jax.experimental.pallas import tpu as pltpu
from jax.experimental.pallas import tpu_sc as plsc

VOCAB = 100000
D = 32
DP = 128
B = 1024
CTX = 20
NC, NS = 2, 16
NW = NC * NS
TV = 2048

_PSEL = np.zeros((DP, D), dtype=np.float32)
_PSEL[:D, :D] = np.eye(D, dtype=np.float32)


def _sc_gather(emb128, idx):
    n = idx.shape[0]
    per_w = n // NW
    mesh = plsc.VectorSubcoreMesh(core_axis_name="c", subcore_axis_name="s")

    @functools.partial(
        pl.kernel, mesh=mesh,
        out_type=jax.ShapeDtypeStruct((n, DP), emb128.dtype),
        scratch_types=[
            pltpu.VMEM((per_w,), jnp.int32),
            pltpu.VMEM((per_w, DP), jnp.float32),
            pltpu.SemaphoreType.DMA,
        ],
    )
    def gather_kernel(emb_hbm, idx_hbm, out_hbm, idx_v, rows_v, sem):
        wid = lax.axis_index("s") * NC + lax.axis_index("c")
        base = wid * per_w
        pltpu.sync_copy(idx_hbm.at[pl.ds(base, per_w)], idx_v)
        pltpu.async_copy(emb_hbm.at[idx_v], rows_v, sem).wait()
        pltpu.sync_copy(rows_v, out_hbm.at[pl.ds(base, per_w)])

    return gather_kernel(emb128, idx)


def _proj_kernel(g_ref, p_ref, w_ref, b_ref, out_ref, pooled_ref):
    @pl.when(pl.program_id(0) == 0)
    def _():
        acc = g_ref[0]
        for c in range(1, CTX):
            acc = acc + g_ref[c]
        pooled_ref[:, :D] = (jax.lax.dot_general(
            acc, p_ref[...], (((1,), (0,)), ((), ())),
            preferred_element_type=jnp.float32,
            precision=jax.lax.Precision.HIGHEST) / CTX).astype(jnp.bfloat16)
        pooled_ref[:, D:] = jnp.ones((B, 1), jnp.bfloat16)
    w_aug = jnp.concatenate([w_ref[...], b_ref[...]], axis=0).astype(jnp.bfloat16)
    out_ref[...] = jax.lax.dot_general(
        w_aug, pooled_ref[...], (((0,), (1,)), ((), ())),
        preferred_element_type=jnp.float32)


def kernel(inputs, emb, W, b):
    idx = inputs.T.reshape(B * CTX).astype(jnp.int32)
    emb128 = jnp.pad(emb, ((0, 0), (0, DP - D)))
    gathered = _sc_gather(emb128, idx)
    g3 = gathered.reshape(CTX, B, DP)
    psel = jnp.asarray(_PSEL)
    wt = W.T
    b2d = b.reshape(1, VOCAB)
    nv = pl.cdiv(VOCAB, TV)
    out_t = pl.pallas_call(
        _proj_kernel,
        grid=(nv,),
        in_specs=[
            pl.BlockSpec((CTX, B, DP), lambda i: (0, 0, 0)),
            pl.BlockSpec((DP, D), lambda i: (0, 0)),
            pl.BlockSpec((D, TV), lambda i: (0, i)),
            pl.BlockSpec((1, TV), lambda i: (0, i)),
        ],
        out_specs=pl.BlockSpec((TV, B), lambda i: (i, 0)),
        out_shape=jax.ShapeDtypeStruct((VOCAB, B), jnp.float32),
        scratch_shapes=[pltpu.VMEM((B, D + 1), jnp.bfloat16)],
    )(g3, psel, wt, b2d)
    return out_t.T
